# Initial kernel scaffold; baseline (speedup 1.0000x reference)
import jax
import jax.numpy as jnp
from jax.experimental import pallas as pl


def kernel(x, edge_index, edge_in, edge_out, query_edges, in_w, out_w, W1, W2, W3, b1, b2, b3, Wl, bl):
    raise NotImplementedError("write your pallas kernel here")


# speedup vs baseline: 2.6426x; 2.6426x over previous
import jax
import jax.numpy as jnp
from jax.experimental import pallas as pl


def _coefs(ei, ew, n):
    e = ei.shape[1]
    w = ew if ew is not None else jnp.ones((e,), jnp.float32)
    deg = jax.ops.segment_sum(w, ei[1], num_segments=n) + 1.0
    dis = deg ** -0.5
    return dis


def _conv(h, ei, ew, dis):
    e = ei.shape[1]
    w = ew if ew is not None else jnp.ones((e,), jnp.float32)
    hp = dis[:, None] * h
    msg = w[:, None] * jnp.take(hp, ei[0], axis=0)
    acc = jax.ops.segment_sum(msg, ei[1], num_segments=h.shape[0])
    return dis[:, None] * (acc + hp)


def kernel(x, edge_index, edge_in, edge_out, query_edges, in_w, out_w,
           W1, W2, W3, b1, b2, b3, Wl, bl):
    n = x.shape[0]
    H = W1.shape[0]
    dis1 = _coefs(edge_index, None, n)
    dis2 = _coefs(edge_in, in_w, n)
    dis3 = _coefs(edge_out, out_w, n)

    h = x @ W1.T
    for (W, b) in ((W2, b1), (W3, b2)):
        g1 = jax.nn.relu(_conv(h, edge_index, None, dis1) + b)
        g2 = jax.nn.relu(_conv(h, edge_in, in_w, dis2) + b)
        g3 = jax.nn.relu(_conv(h, edge_out, out_w, dis3) + b)
        h = g1 @ W[:, :H].T + g2 @ W[:, H:2 * H].T + g3 @ W[:, 2 * H:].T
    g1 = jax.nn.relu(_conv(h, edge_index, None, dis1) + b3)
    g2 = jax.nn.relu(_conv(h, edge_in, in_w, dis2) + b3)
    g3 = jax.nn.relu(_conv(h, edge_out, out_w, dis3) + b3)

    A, B = Wl[:, :3 * H], Wl[:, 3 * H:]
    P = g1 @ A[:, :H].T + g2 @ A[:, H:2 * H].T + g3 @ A[:, 2 * H:].T
    Qm = g1 @ B[:, :H].T + g2 @ B[:, H:2 * H].T + g3 @ B[:, 2 * H:].T
    logits = (jnp.take(P, query_edges[:, 0], axis=0)
              + jnp.take(Qm, query_edges[:, 1], axis=0) + bl)
    return jax.nn.softmax(logits, axis=1)


# speedup vs baseline: 7.3628x; 2.7862x over previous
import dataclasses
import functools

import jax
import jax.numpy as jnp
from jax import lax
from jax.experimental import pallas as pl
from jax.experimental.pallas import tpu as pltpu
from jax.experimental.pallas import tpu_sc as plsc

NC = 2
NS = 16
CH = 128


def _sc_params():
    cp = pltpu.CompilerParams()
    if "needs_layout_passes" in pltpu.CompilerParams.__dataclass_fields__:
        cp = dataclasses.replace(cp, needs_layout_passes=False)
    return cp


def _sc_conv(hp_flat, rows, cols, wts, n):
    ec = rows.shape[1]
    half = ec // 2
    rps = n // NS
    mesh = plsc.VectorSubcoreMesh(core_axis_name="c", subcore_axis_name="s")

    @functools.partial(
        pl.kernel, mesh=mesh,
        out_type=jax.ShapeDtypeStruct((NC, 3, n, 128), jnp.float32),
        scratch_types=[
            pltpu.VMEM((1, CH), jnp.int32),
            pltpu.VMEM((1, CH), jnp.int32),
            pltpu.VMEM((1, CH), jnp.float32),
            pltpu.VMEM((CH, 128), jnp.float32),
            pltpu.VMEM((128, 128), jnp.float32),
            pltpu.VMEM_SHARED((n, 128), jnp.float32),
            pltpu.SemaphoreType.DMA,
        ],
        compiler_params=_sc_params(),
    )
    def k(hp_hbm, r_hbm, c_hbm, w_hbm, out_hbm,
          ridx, cidx, wv, rows_v, zbuf, acc, sem):
        cid = lax.axis_index("c")
        sid = lax.axis_index("s")
        zv = jnp.zeros((16,), jnp.float32)

        @pl.loop(0, 128)
        def _(r):
            for l in range(8):
                zbuf[r, pl.ds(l * 16, 16)] = zv

        for t in range(3):
            base = sid * rps
            for j in range(rps // 128):
                pltpu.sync_copy(zbuf, acc.at[pl.ds(base + j * 128, 128)])
            plsc.subcore_barrier()

            lo = cid * half + sid

            @pl.loop(lo, (cid + 1) * half, step=NS)
            def _(ch):
                pltpu.sync_copy(r_hbm.at[t, ch], ridx)
                pltpu.sync_copy(c_hbm.at[t, ch], cidx)
                pltpu.async_copy(hp_hbm.at[ridx.at[0]], rows_v, sem).wait()
                if t > 0:
                    pltpu.sync_copy(w_hbm.at[t, ch], wv)

                    @pl.loop(0, CH)
                    def _(j):
                        b = plsc.load_gather(
                            wv, [jnp.zeros((16,), jnp.int32),
                                 jnp.full((16,), j, jnp.int32)])
                        for l in range(8):
                            sl = pl.ds(l * 16, 16)
                            rows_v[j, sl] = rows_v[j, sl] * b
                pltpu.sync_copy(rows_v, acc.at[cidx.at[0]], add=True)

            plsc.subcore_barrier()
            pltpu.sync_copy(acc.at[pl.ds(base, rps)],
                            out_hbm.at[cid, t, pl.ds(base, rps)])
            plsc.subcore_barrier()

    return k(hp_flat, rows, cols, wts)


def _sc_qgather(tab, qidx, qp):
    qc = qidx.shape[1]
    mesh = plsc.VectorSubcoreMesh(core_axis_name="c", subcore_axis_name="s")

    @functools.partial(
        pl.kernel, mesh=mesh,
        out_type=jax.ShapeDtypeStruct((2, qp, 128), jnp.float32),
        scratch_types=[
            pltpu.VMEM((1, CH), jnp.int32),
            pltpu.VMEM((CH, 128), jnp.float32),
            pltpu.SemaphoreType.DMA,
        ],
    )
    def k(tab_hbm, q_hbm, out_hbm, ridx, gv, sem):
        wid = lax.axis_index("s") * NC + lax.axis_index("c")
        for q in range(2):

            @pl.loop(wid, qc, step=NC * NS)
            def _(ch):
                pltpu.sync_copy(q_hbm.at[q, ch], ridx)
                pltpu.async_copy(tab_hbm.at[ridx.at[0]], gv, sem).wait()
                pltpu.sync_copy(gv, out_hbm.at[q, pl.ds(ch * CH, CH)])

    return k(tab, qidx)


def _tc_project(x, w1t, dis, n, bn):
    def body(x_ref, w_ref, d_ref, o_ref):
        h = jnp.dot(x_ref[...], w_ref[...], preferred_element_type=jnp.float32)
        for t in range(3):
            o_ref[t] = d_ref[:, t:t + 1] * h

    return pl.pallas_call(
        body,
        grid=(n // bn,),
        in_specs=[
            pl.BlockSpec((bn, 128), lambda i: (i, 0)),
            pl.BlockSpec((128, 128), lambda i: (0, 0)),
            pl.BlockSpec((bn, 3), lambda i: (i, 0)),
        ],
        out_specs=pl.BlockSpec((3, bn, 128), lambda i: (0, i, 0)),
        out_shape=jax.ShapeDtypeStruct((3, n, 128), jnp.float32),
    )(x, w1t, dis)


def _tc_combine(part, hp, dis, b, wstack, n, bn):
    def body(p_ref, h_ref, d_ref, b_ref, w_ref, o_ref):
        hnew = jnp.zeros((bn, 128), jnp.float32)
        for t in range(3):
            conv = d_ref[:, t:t + 1] * (p_ref[0, t] + p_ref[1, t] + h_ref[t])
            g = jax.nn.relu(conv + b_ref[...])
            hnew = hnew + jnp.dot(g, w_ref[t],
                                  preferred_element_type=jnp.float32)
        for t in range(3):
            o_ref[t] = d_ref[:, t:t + 1] * hnew

    return pl.pallas_call(
        body,
        grid=(n // bn,),
        in_specs=[
            pl.BlockSpec((NC, 3, bn, 128), lambda i: (0, 0, i, 0)),
            pl.BlockSpec((3, bn, 128), lambda i: (0, i, 0)),
            pl.BlockSpec((bn, 3), lambda i: (i, 0)),
            pl.BlockSpec((1, 128), lambda i: (0, 0)),
            pl.BlockSpec((3, 128, 128), lambda i: (0, 0, 0)),
        ],
        out_specs=pl.BlockSpec((3, bn, 128), lambda i: (0, i, 0)),
        out_shape=jax.ShapeDtypeStruct((3, n, 128), jnp.float32),
    )(part, hp, dis, b, wstack)


def _tc_headtab(part, hp, dis, b, whead, n, bn):
    def body(p_ref, h_ref, d_ref, b_ref, w_ref, o_ref):
        pq = jnp.zeros((bn, 8), jnp.float32)
        for t in range(3):
            conv = d_ref[:, t:t + 1] * (p_ref[0, t] + p_ref[1, t] + h_ref[t])
            g = jax.nn.relu(conv + b_ref[...])
            pq = pq + jnp.dot(g, w_ref[t],
                              preferred_element_type=jnp.float32)
        o_ref[...] = jnp.concatenate(
            [pq, jnp.zeros((bn, 120), jnp.float32)], axis=1)

    return pl.pallas_call(
        body,
        grid=(n // bn,),
        in_specs=[
            pl.BlockSpec((NC, 3, bn, 128), lambda i: (0, 0, i, 0)),
            pl.BlockSpec((3, bn, 128), lambda i: (0, i, 0)),
            pl.BlockSpec((bn, 3), lambda i: (i, 0)),
            pl.BlockSpec((1, 128), lambda i: (0, 0)),
            pl.BlockSpec((3, 128, 8), lambda i: (0, 0, 0)),
        ],
        out_specs=pl.BlockSpec((bn, 128), lambda i: (i, 0)),
        out_shape=jax.ShapeDtypeStruct((n, 128), jnp.float32),
    )(part, hp, dis, b, whead)


def _tc_head(g, blv, qp, bq):
    def body(g_ref, b_ref, o_ref):
        lg = g_ref[0, :, 0:3] + g_ref[1, :, 3:6] + b_ref[...]
        m = jnp.max(lg, axis=1, keepdims=True)
        e = jnp.exp(lg - m)
        o_ref[...] = e / jnp.sum(e, axis=1, keepdims=True)

    return pl.pallas_call(
        body,
        grid=(qp // bq,),
        in_specs=[
            pl.BlockSpec((2, bq, 128), lambda i: (0, i, 0)),
            pl.BlockSpec((1, 3), lambda i: (0, 0)),
        ],
        out_specs=pl.BlockSpec((bq, 3), lambda i: (i, 0)),
        out_shape=jax.ShapeDtypeStruct((qp, 3), jnp.float32),
    )(g, blv)


def kernel(x, edge_index, edge_in, edge_out, query_edges, in_w, out_w,
           W1, W2, W3, b1, b2, b3, Wl, bl):
    n, d = x.shape
    h = W1.shape[0]
    e = edge_index.shape[1]
    q = query_edges.shape[0]
    ec = e // CH
    npad = ((n + NS * 8 - 1) // (NS * 8)) * (NS * 8)
    bn = 2048

    ones = jnp.ones((e,), jnp.float32)
    srcs = [edge_index[0], edge_in[0], edge_out[0]]
    dsts = [edge_index[1], edge_in[1], edge_out[1]]
    ws = [ones, in_w, out_w]

    dis_cols = []
    for t in range(3):
        deg = jax.ops.segment_sum(ws[t], dsts[t], num_segments=n) + 1.0
        dis_cols.append(deg ** -0.5)
    dis = jnp.pad(jnp.stack(dis_cols, axis=1), ((0, npad - n), (0, 0)))
    x = jnp.pad(x, ((0, npad - n), (0, 0)))

    rows = jnp.stack([srcs[t] + t * npad for t in range(3)])
    cols = jnp.stack(dsts)
    wts = jnp.stack(ws)
    rows = rows.reshape(3, ec, 1, CH).astype(jnp.int32)
    cols = cols.reshape(3, ec, 1, CH).astype(jnp.int32)
    wts = wts.reshape(3, ec, 1, CH)

    qp = ((q + CH - 1) // CH) * CH
    qpad = jnp.pad(query_edges, ((0, qp - q), (0, 0)))
    qidx = qpad.T.reshape(2, qp // CH, 1, CH).astype(jnp.int32)

    w1t = W1.T
    w2s = jnp.stack([W2[:, t * h:(t + 1) * h].T for t in range(3)])
    w3s = jnp.stack([W3[:, t * h:(t + 1) * h].T for t in range(3)])
    a, bheadm = Wl[:, :3 * h], Wl[:, 3 * h:]
    whead = jnp.stack([
        jnp.concatenate([a[:, t * h:(t + 1) * h].T,
                         bheadm[:, t * h:(t + 1) * h].T,
                         jnp.zeros((h, 2), jnp.float32)], axis=1)
        for t in range(3)])

    hp = _tc_project(x, w1t, dis, npad, bn)
    part = _sc_conv(hp.reshape(3 * npad, 128), rows, cols, wts, npad)
    hp = _tc_combine(part, hp, dis, b1, w2s, npad, bn)
    part = _sc_conv(hp.reshape(3 * npad, 128), rows, cols, wts, npad)
    hp = _tc_combine(part, hp, dis, b2, w3s, npad, bn)
    part = _sc_conv(hp.reshape(3 * npad, 128), rows, cols, wts, npad)
    tab = _tc_headtab(part, hp, dis, b3, whead, npad, bn)
    g = _sc_qgather(tab, qidx, qp)
    out = _tc_head(g, bl.reshape(1, 3), qp, qp // 8)
    return out[:q]
